# baseline (speedup 1.0000x reference)
import jax
import jax.numpy as jnp
from jax.experimental import pallas as pl
from jax.experimental.pallas import tpu as pltpu

EPS = 1e-5


def _p1_proj(z_ref, wcat_ref, got_ref, gb_ref, b_ref, gbo_ref, lnw_ref, lnb_ref,
             abcd_ref, gate_ref):
    x = z_ref[...]
    mu = jnp.mean(x, axis=-1, keepdims=True)
    m2 = jnp.mean(x * x, axis=-1, keepdims=True)
    var = m2 - mu * mu
    zn = (x - mu) * jax.lax.rsqrt(var + EPS) * lnw_ref[...] + lnb_ref[...]
    znb = zn.astype(jnp.bfloat16)

    acc = jax.lax.dot_general(
        wcat_ref[...], znb, (((1,), (1,)), ((), ())),
        preferred_element_type=jnp.float32)
    n_rep = acc.shape[1] // 128
    gb = jnp.concatenate([gb_ref[...]] * n_rep, axis=1)
    bb = jnp.concatenate([b_ref[...]] * n_rep, axis=1)
    half = gb.shape[0]
    y = jax.nn.sigmoid(acc[:half] + gb) * (acc[half:] + bb)
    abcd_ref[...] = y.astype(jnp.bfloat16)

    g = jnp.dot(znb, got_ref[...], preferred_element_type=jnp.float32)
    gate_ref[...] = g.astype(jnp.bfloat16)


def _p2_einsum(a_ref, b_ref, c_ref, d_ref, k_ref):
    dn = (((1,), (1,)), ((), ()))
    for c in range(a_ref.shape[0]):
        p = jax.lax.dot_general(a_ref[c], b_ref[c], dn,
                                preferred_element_type=jnp.float32)
        q = jax.lax.dot_general(c_ref[c], d_ref[c], dn,
                                preferred_element_type=jnp.float32)
        k_ref[c] = (p * q).astype(jnp.bfloat16)


def _p3_out(k_ref, gate_ref, wo2_ref, bo2_ref, gbo_ref, o_ref):
    kt = k_ref[...].astype(jnp.float32)
    mu = jnp.mean(kt, axis=0, keepdims=True)
    kc = kt - mu
    var = jnp.mean(kc * kc, axis=0, keepdims=True)
    kn = (kc * jax.lax.rsqrt(var + EPS)).astype(jnp.bfloat16)
    o = jax.lax.dot_general(kn, wo2_ref[...], (((0,), (0,)), ((), ())),
                            preferred_element_type=jnp.float32)
    gate = jax.nn.sigmoid(gate_ref[...].astype(jnp.float32) + gbo_ref[...])
    o_ref[...] = (o + bo2_ref[...]) * gate


def kernel(z, Wa, ba, Ga, gba, Wb, bb, Gb, gbb, Wc, bc, Gc, gbc,
           Wd, bd, Gd, gbd, ln_in_w, ln_in_b, ln_o_w, ln_o_b, Go, gbo, Wo, bo):
    B, N, _, D = z.shape
    C = Wa.shape[0]
    NN = N * N
    TM = 1024
    TQ = 8192

    z2 = z.reshape(NN, D)
    Wcat = jnp.concatenate([Ga, Gb, Gc, Gd, Wa, Wb, Wc, Wd], axis=0).astype(jnp.bfloat16)
    GoT = Go.T.astype(jnp.bfloat16)
    gbcat = jnp.concatenate([gba, gbb, gbc, gbd])
    bcat = jnp.concatenate([ba, bb, bc, bd])
    gb_tile = jnp.broadcast_to(gbcat[:, None], (4 * C, 128))
    b_tile = jnp.broadcast_to(bcat[:, None], (4 * C, 128))
    gbo2 = gbo.reshape(1, D)
    lnw2 = ln_in_w.reshape(1, D)
    lnb2 = ln_in_b.reshape(1, D)

    abcd, gate = pl.pallas_call(
        _p1_proj,
        grid=(NN // TM,),
        in_specs=[
            pl.BlockSpec((TM, D), lambda i: (i, 0)),
            pl.BlockSpec((8 * C, D), lambda i: (0, 0)),
            pl.BlockSpec((D, D), lambda i: (0, 0)),
            pl.BlockSpec((4 * C, 128), lambda i: (0, 0)),
            pl.BlockSpec((4 * C, 128), lambda i: (0, 0)),
            pl.BlockSpec((1, D), lambda i: (0, 0)),
            pl.BlockSpec((1, D), lambda i: (0, 0)),
            pl.BlockSpec((1, D), lambda i: (0, 0)),
        ],
        out_specs=[
            pl.BlockSpec((4 * C, TM), lambda i: (0, i)),
            pl.BlockSpec((TM, D), lambda i: (i, 0)),
        ],
        out_shape=[
            jax.ShapeDtypeStruct((4 * C, NN), jnp.bfloat16),
            jax.ShapeDtypeStruct((NN, D), jnp.bfloat16),
        ],
        compiler_params=pltpu.CompilerParams(
            dimension_semantics=("arbitrary",)),
        name="qeu_proj",
    )(z2, Wcat, GoT, gb_tile, b_tile, gbo2, lnw2, lnb2)

    CB = 16
    abcd3 = abcd.reshape(4 * C, N, N)
    kt = pl.pallas_call(
        _p2_einsum,
        grid=(C // CB,),
        in_specs=[
            pl.BlockSpec((CB, N, N), lambda o: (o, 0, 0)),
            pl.BlockSpec((CB, N, N), lambda o: (o + C // CB, 0, 0)),
            pl.BlockSpec((CB, N, N), lambda o: (o + 2 * (C // CB), 0, 0)),
            pl.BlockSpec((CB, N, N), lambda o: (o + 3 * (C // CB), 0, 0)),
        ],
        out_specs=pl.BlockSpec((CB, N, N), lambda o: (o, 0, 0)),
        out_shape=jax.ShapeDtypeStruct((C, N, N), jnp.bfloat16),
        compiler_params=pltpu.CompilerParams(
            dimension_semantics=("arbitrary",)),
        name="qeu_einsum",
    )(abcd3, abcd3, abcd3, abcd3)

    Wo2 = (Wo.T * ln_o_w[:, None]).astype(jnp.bfloat16)
    bo2 = (bo + ln_o_b @ Wo.T).reshape(1, D)

    k2 = kt.reshape(C, NN)
    out2 = pl.pallas_call(
        _p3_out,
        grid=(NN // TQ,),
        in_specs=[
            pl.BlockSpec((C, TQ), lambda i: (0, i)),
            pl.BlockSpec((TQ, D), lambda i: (i, 0)),
            pl.BlockSpec((C, D), lambda i: (0, 0)),
            pl.BlockSpec((1, D), lambda i: (0, 0)),
            pl.BlockSpec((1, D), lambda i: (0, 0)),
        ],
        out_specs=pl.BlockSpec((TQ, D), lambda i: (i, 0)),
        out_shape=jax.ShapeDtypeStruct((NN, D), jnp.float32),
        compiler_params=pltpu.CompilerParams(
            dimension_semantics=("arbitrary",)),
        name="qeu_out",
    )(k2, gate, Wo2, bo2, gbo2)

    return out2.reshape(B, N, N, D)


# speedup vs baseline: 1.0815x; 1.0815x over previous
import jax
import jax.numpy as jnp
from jax.experimental import pallas as pl
from jax.experimental.pallas import tpu as pltpu

EPS = 1e-5


def _p1_proj(z_ref, wcat_ref, got_ref, gb_ref, b_ref, gbo_ref, lnw_ref, lnb_ref,
             abcd_ref, gate_ref):
    x = z_ref[...]
    mu = jnp.mean(x, axis=-1, keepdims=True)
    m2 = jnp.mean(x * x, axis=-1, keepdims=True)
    var = m2 - mu * mu
    zn = (x - mu) * jax.lax.rsqrt(var + EPS) * lnw_ref[...] + lnb_ref[...]
    znb = zn.astype(jnp.bfloat16)

    acc = jax.lax.dot_general(
        wcat_ref[...], znb, (((1,), (1,)), ((), ())),
        preferred_element_type=jnp.float32)
    n_rep = acc.shape[1] // 128
    gb = jnp.concatenate([gb_ref[...]] * n_rep, axis=1)
    bb = jnp.concatenate([b_ref[...]] * n_rep, axis=1)
    half = gb.shape[0]
    y = jax.nn.sigmoid(acc[:half] + gb) * (acc[half:] + bb)
    abcd_ref[...] = y.astype(jnp.bfloat16)

    g = jnp.dot(znb, got_ref[...], preferred_element_type=jnp.float32)
    gate_ref[...] = g.astype(jnp.bfloat16)


def _p2_einsum(a_ref, b_ref, c_ref, d_ref, k_ref):
    dn = (((1,), (1,)), ((), ()))
    for c in range(a_ref.shape[0]):
        p = jax.lax.dot_general(a_ref[c], b_ref[c], dn,
                                preferred_element_type=jnp.float32)
        q = jax.lax.dot_general(c_ref[c], d_ref[c], dn,
                                preferred_element_type=jnp.float32)
        k_ref[c] = (p * q).astype(jnp.bfloat16)


def _p3_out(k_ref, gate_ref, wo2_ref, bo2_ref, gbo_ref, o_ref):
    kt = k_ref[...].astype(jnp.float32)
    mu = jnp.mean(kt, axis=0, keepdims=True)
    kc = kt - mu
    var = jnp.mean(kc * kc, axis=0, keepdims=True)
    kn = (kc * jax.lax.rsqrt(var + EPS)).astype(jnp.bfloat16)
    o = jax.lax.dot_general(kn, wo2_ref[...], (((0,), (0,)), ((), ())),
                            preferred_element_type=jnp.float32)
    gate = jax.nn.sigmoid(gate_ref[...].astype(jnp.float32) + gbo_ref[...])
    o_ref[...] = (o + bo2_ref[...]) * gate


def kernel(z, Wa, ba, Ga, gba, Wb, bb, Gb, gbb, Wc, bc, Gc, gbc,
           Wd, bd, Gd, gbd, ln_in_w, ln_in_b, ln_o_w, ln_o_b, Go, gbo, Wo, bo):
    B, N, _, D = z.shape
    C = Wa.shape[0]
    NN = N * N
    TM = 2048
    TQ = 8192

    z2 = z.reshape(NN, D)
    Wcat = jnp.concatenate([Ga, Gb, Gc, Gd, Wa, Wb, Wc, Wd], axis=0).astype(jnp.bfloat16)
    GoT = Go.T.astype(jnp.bfloat16)
    gbcat = jnp.concatenate([gba, gbb, gbc, gbd])
    bcat = jnp.concatenate([ba, bb, bc, bd])
    gb_tile = jnp.broadcast_to(gbcat[:, None], (4 * C, 128))
    b_tile = jnp.broadcast_to(bcat[:, None], (4 * C, 128))
    gbo2 = gbo.reshape(1, D)
    lnw2 = ln_in_w.reshape(1, D)
    lnb2 = ln_in_b.reshape(1, D)

    abcd, gate = pl.pallas_call(
        _p1_proj,
        grid=(NN // TM,),
        in_specs=[
            pl.BlockSpec((TM, D), lambda i: (i, 0)),
            pl.BlockSpec((8 * C, D), lambda i: (0, 0)),
            pl.BlockSpec((D, D), lambda i: (0, 0)),
            pl.BlockSpec((4 * C, 128), lambda i: (0, 0)),
            pl.BlockSpec((4 * C, 128), lambda i: (0, 0)),
            pl.BlockSpec((1, D), lambda i: (0, 0)),
            pl.BlockSpec((1, D), lambda i: (0, 0)),
            pl.BlockSpec((1, D), lambda i: (0, 0)),
        ],
        out_specs=[
            pl.BlockSpec((4 * C, TM), lambda i: (0, i)),
            pl.BlockSpec((TM, D), lambda i: (i, 0)),
        ],
        out_shape=[
            jax.ShapeDtypeStruct((4 * C, NN), jnp.bfloat16),
            jax.ShapeDtypeStruct((NN, D), jnp.bfloat16),
        ],
        compiler_params=pltpu.CompilerParams(
            dimension_semantics=("arbitrary",)),
        name="qeu_proj",
    )(z2, Wcat, GoT, gb_tile, b_tile, gbo2, lnw2, lnb2)

    CB = 16
    abcd3 = abcd.reshape(4 * C, N, N)
    kt = pl.pallas_call(
        _p2_einsum,
        grid=(C // CB,),
        in_specs=[
            pl.BlockSpec((CB, N, N), lambda o: (o, 0, 0)),
            pl.BlockSpec((CB, N, N), lambda o: (o + C // CB, 0, 0)),
            pl.BlockSpec((CB, N, N), lambda o: (o + 2 * (C // CB), 0, 0)),
            pl.BlockSpec((CB, N, N), lambda o: (o + 3 * (C // CB), 0, 0)),
        ],
        out_specs=pl.BlockSpec((CB, N, N), lambda o: (o, 0, 0)),
        out_shape=jax.ShapeDtypeStruct((C, N, N), jnp.bfloat16),
        compiler_params=pltpu.CompilerParams(
            dimension_semantics=("arbitrary",)),
        name="qeu_einsum",
    )(abcd3, abcd3, abcd3, abcd3)

    Wo2 = (Wo.T * ln_o_w[:, None]).astype(jnp.bfloat16)
    bo2 = (bo + ln_o_b @ Wo.T).reshape(1, D)

    k2 = kt.reshape(C, NN)
    out2 = pl.pallas_call(
        _p3_out,
        grid=(NN // TQ,),
        in_specs=[
            pl.BlockSpec((C, TQ), lambda i: (0, i)),
            pl.BlockSpec((TQ, D), lambda i: (i, 0)),
            pl.BlockSpec((C, D), lambda i: (0, 0)),
            pl.BlockSpec((1, D), lambda i: (0, 0)),
            pl.BlockSpec((1, D), lambda i: (0, 0)),
        ],
        out_specs=pl.BlockSpec((TQ, D), lambda i: (i, 0)),
        out_shape=jax.ShapeDtypeStruct((NN, D), jnp.float32),
        compiler_params=pltpu.CompilerParams(
            dimension_semantics=("arbitrary",)),
        name="qeu_out",
    )(k2, gate, Wo2, bo2, gbo2)

    return out2.reshape(B, N, N, D)


# speedup vs baseline: 1.0842x; 1.0026x over previous
import jax
import jax.numpy as jnp
from jax.experimental import pallas as pl
from jax.experimental.pallas import tpu as pltpu

EPS = 1e-5


def _p1_proj(z_ref, wcat_ref, got_ref, gb_ref, b_ref, gbo_ref, lnw_ref, lnb_ref,
             abcd_ref, gate_ref):
    x = z_ref[...]
    mu = jnp.mean(x, axis=-1, keepdims=True)
    m2 = jnp.mean(x * x, axis=-1, keepdims=True)
    var = m2 - mu * mu
    zn = (x - mu) * jax.lax.rsqrt(var + EPS) * lnw_ref[...] + lnb_ref[...]
    znb = zn.astype(jnp.bfloat16)

    acc = jax.lax.dot_general(
        wcat_ref[...], znb, (((1,), (1,)), ((), ())),
        preferred_element_type=jnp.float32)
    n_rep = acc.shape[1] // 128
    gb = jnp.concatenate([gb_ref[...]] * n_rep, axis=1)
    bb = jnp.concatenate([b_ref[...]] * n_rep, axis=1)
    half = gb.shape[0]
    y = jax.nn.sigmoid(acc[:half] + gb) * (acc[half:] + bb)
    abcd_ref[...] = y.astype(jnp.bfloat16)

    g = jnp.dot(znb, got_ref[...], preferred_element_type=jnp.float32)
    gate_ref[...] = g.astype(jnp.bfloat16)


def _p2_einsum(a_ref, b_ref, c_ref, d_ref, k_ref):
    dn = (((1,), (1,)), ((), ()))
    for c in range(a_ref.shape[0]):
        p = jax.lax.dot_general(a_ref[c], b_ref[c], dn,
                                preferred_element_type=jnp.float32)
        q = jax.lax.dot_general(c_ref[c], d_ref[c], dn,
                                preferred_element_type=jnp.float32)
        k_ref[c] = (p * q).astype(jnp.bfloat16)


def _p3_out(k_ref, gate_ref, wo2_ref, bo2_ref, gbo_ref, o_ref):
    kt = k_ref[...].astype(jnp.float32)
    mu = jnp.mean(kt, axis=0, keepdims=True)
    kc = kt - mu
    var = jnp.mean(kc * kc, axis=0, keepdims=True)
    kn = (kc * jax.lax.rsqrt(var + EPS)).astype(jnp.bfloat16)
    o = jax.lax.dot_general(kn, wo2_ref[...], (((0,), (0,)), ((), ())),
                            preferred_element_type=jnp.float32)
    gate = jax.nn.sigmoid(gate_ref[...].astype(jnp.float32) + gbo_ref[...])
    o_ref[...] = (o + bo2_ref[...]) * gate


def kernel(z, Wa, ba, Ga, gba, Wb, bb, Gb, gbb, Wc, bc, Gc, gbc,
           Wd, bd, Gd, gbd, ln_in_w, ln_in_b, ln_o_w, ln_o_b, Go, gbo, Wo, bo):
    B, N, _, D = z.shape
    C = Wa.shape[0]
    NN = N * N
    TM = 2048
    TQ = 16384

    z2 = z.reshape(NN, D)
    Wcat = jnp.concatenate([Ga, Gb, Gc, Gd, Wa, Wb, Wc, Wd], axis=0).astype(jnp.bfloat16)
    GoT = Go.T.astype(jnp.bfloat16)
    gbcat = jnp.concatenate([gba, gbb, gbc, gbd])
    bcat = jnp.concatenate([ba, bb, bc, bd])
    gb_tile = jnp.broadcast_to(gbcat[:, None], (4 * C, 128))
    b_tile = jnp.broadcast_to(bcat[:, None], (4 * C, 128))
    gbo2 = gbo.reshape(1, D)
    lnw2 = ln_in_w.reshape(1, D)
    lnb2 = ln_in_b.reshape(1, D)

    abcd, gate = pl.pallas_call(
        _p1_proj,
        grid=(NN // TM,),
        in_specs=[
            pl.BlockSpec((TM, D), lambda i: (i, 0)),
            pl.BlockSpec((8 * C, D), lambda i: (0, 0)),
            pl.BlockSpec((D, D), lambda i: (0, 0)),
            pl.BlockSpec((4 * C, 128), lambda i: (0, 0)),
            pl.BlockSpec((4 * C, 128), lambda i: (0, 0)),
            pl.BlockSpec((1, D), lambda i: (0, 0)),
            pl.BlockSpec((1, D), lambda i: (0, 0)),
            pl.BlockSpec((1, D), lambda i: (0, 0)),
        ],
        out_specs=[
            pl.BlockSpec((4 * C, TM), lambda i: (0, i)),
            pl.BlockSpec((TM, D), lambda i: (i, 0)),
        ],
        out_shape=[
            jax.ShapeDtypeStruct((4 * C, NN), jnp.bfloat16),
            jax.ShapeDtypeStruct((NN, D), jnp.bfloat16),
        ],
        compiler_params=pltpu.CompilerParams(
            dimension_semantics=("arbitrary",)),
        name="qeu_proj",
    )(z2, Wcat, GoT, gb_tile, b_tile, gbo2, lnw2, lnb2)

    CB = 16
    abcd3 = abcd.reshape(4 * C, N, N)
    kt = pl.pallas_call(
        _p2_einsum,
        grid=(C // CB,),
        in_specs=[
            pl.BlockSpec((CB, N, N), lambda o: (o, 0, 0)),
            pl.BlockSpec((CB, N, N), lambda o: (o + C // CB, 0, 0)),
            pl.BlockSpec((CB, N, N), lambda o: (o + 2 * (C // CB), 0, 0)),
            pl.BlockSpec((CB, N, N), lambda o: (o + 3 * (C // CB), 0, 0)),
        ],
        out_specs=pl.BlockSpec((CB, N, N), lambda o: (o, 0, 0)),
        out_shape=jax.ShapeDtypeStruct((C, N, N), jnp.bfloat16),
        compiler_params=pltpu.CompilerParams(
            dimension_semantics=("arbitrary",)),
        name="qeu_einsum",
    )(abcd3, abcd3, abcd3, abcd3)

    Wo2 = (Wo.T * ln_o_w[:, None]).astype(jnp.bfloat16)
    bo2 = (bo + ln_o_b @ Wo.T).reshape(1, D)

    k2 = kt.reshape(C, NN)
    out2 = pl.pallas_call(
        _p3_out,
        grid=(NN // TQ,),
        in_specs=[
            pl.BlockSpec((C, TQ), lambda i: (0, i)),
            pl.BlockSpec((TQ, D), lambda i: (i, 0)),
            pl.BlockSpec((C, D), lambda i: (0, 0)),
            pl.BlockSpec((1, D), lambda i: (0, 0)),
            pl.BlockSpec((1, D), lambda i: (0, 0)),
        ],
        out_specs=pl.BlockSpec((TQ, D), lambda i: (i, 0)),
        out_shape=jax.ShapeDtypeStruct((NN, D), jnp.float32),
        compiler_params=pltpu.CompilerParams(
            dimension_semantics=("arbitrary",)),
        name="qeu_out",
    )(k2, gate, Wo2, bo2, gbo2)

    return out2.reshape(B, N, N, D)


# speedup vs baseline: 1.0978x; 1.0125x over previous
import jax
import jax.numpy as jnp
from jax.experimental import pallas as pl
from jax.experimental.pallas import tpu as pltpu

EPS = 1e-5


def _p1_proj(z_ref, wcat_ref, got_ref, gb_ref, b_ref, gbo_ref, lnw_ref, lnb_ref,
             abcd_ref, gate_ref):
    x = z_ref[...]
    mu = jnp.mean(x, axis=-1, keepdims=True)
    m2 = jnp.mean(x * x, axis=-1, keepdims=True)
    var = m2 - mu * mu
    zn = (x - mu) * jax.lax.rsqrt(var + EPS) * lnw_ref[...] + lnb_ref[...]
    znb = zn.astype(jnp.bfloat16)

    acc = jax.lax.dot_general(
        wcat_ref[...], znb, (((1,), (1,)), ((), ())),
        preferred_element_type=jnp.float32)
    n_rep = acc.shape[1] // 128
    gb = jnp.concatenate([gb_ref[...]] * n_rep, axis=1)
    bb = jnp.concatenate([b_ref[...]] * n_rep, axis=1)
    half = gb.shape[0]
    sig = 0.5 * jnp.tanh(0.5 * (acc[:half] + gb)) + 0.5
    y = sig * (acc[half:] + bb)
    abcd_ref[...] = y.astype(jnp.bfloat16)

    g = jnp.dot(znb, got_ref[...], preferred_element_type=jnp.float32)
    gate_ref[...] = g.astype(jnp.bfloat16)


def _p2_einsum(a_ref, b_ref, c_ref, d_ref, k_ref):
    dn = (((1,), (1,)), ((), ()))
    for c in range(a_ref.shape[0]):
        p = jax.lax.dot_general(a_ref[c], b_ref[c], dn,
                                preferred_element_type=jnp.float32)
        q = jax.lax.dot_general(c_ref[c], d_ref[c], dn,
                                preferred_element_type=jnp.float32)
        k_ref[c] = (p * q).astype(jnp.bfloat16)


def _p3_out(k_ref, gate_ref, wo2_ref, bo2_ref, gbo_ref, o_ref):
    kt = k_ref[...].astype(jnp.float32)
    mu = jnp.mean(kt, axis=0, keepdims=True)
    kc = kt - mu
    var = jnp.mean(kc * kc, axis=0, keepdims=True)
    kn = (kc * jax.lax.rsqrt(var + EPS)).astype(jnp.bfloat16)
    o = jax.lax.dot_general(kn, wo2_ref[...], (((0,), (0,)), ((), ())),
                            preferred_element_type=jnp.float32)
    gate = jax.nn.sigmoid(gate_ref[...].astype(jnp.float32) + gbo_ref[...])
    o_ref[...] = (o + bo2_ref[...]) * gate


def kernel(z, Wa, ba, Ga, gba, Wb, bb, Gb, gbb, Wc, bc, Gc, gbc,
           Wd, bd, Gd, gbd, ln_in_w, ln_in_b, ln_o_w, ln_o_b, Go, gbo, Wo, bo):
    B, N, _, D = z.shape
    C = Wa.shape[0]
    NN = N * N
    TM = 2048
    TQ = 16384

    z2 = z.reshape(NN, D)
    Wcat = jnp.concatenate([Ga, Gb, Gc, Gd, Wa, Wb, Wc, Wd], axis=0).astype(jnp.bfloat16)
    GoT = Go.T.astype(jnp.bfloat16)
    gbcat = jnp.concatenate([gba, gbb, gbc, gbd])
    bcat = jnp.concatenate([ba, bb, bc, bd])
    gb_tile = jnp.broadcast_to(gbcat[:, None], (4 * C, 128))
    b_tile = jnp.broadcast_to(bcat[:, None], (4 * C, 128))
    gbo2 = gbo.reshape(1, D)
    lnw2 = ln_in_w.reshape(1, D)
    lnb2 = ln_in_b.reshape(1, D)

    abcd, gate = pl.pallas_call(
        _p1_proj,
        grid=(NN // TM,),
        in_specs=[
            pl.BlockSpec((TM, D), lambda i: (i, 0)),
            pl.BlockSpec((8 * C, D), lambda i: (0, 0)),
            pl.BlockSpec((D, D), lambda i: (0, 0)),
            pl.BlockSpec((4 * C, 128), lambda i: (0, 0)),
            pl.BlockSpec((4 * C, 128), lambda i: (0, 0)),
            pl.BlockSpec((1, D), lambda i: (0, 0)),
            pl.BlockSpec((1, D), lambda i: (0, 0)),
            pl.BlockSpec((1, D), lambda i: (0, 0)),
        ],
        out_specs=[
            pl.BlockSpec((4 * C, TM), lambda i: (0, i)),
            pl.BlockSpec((TM, D), lambda i: (i, 0)),
        ],
        out_shape=[
            jax.ShapeDtypeStruct((4 * C, NN), jnp.bfloat16),
            jax.ShapeDtypeStruct((NN, D), jnp.bfloat16),
        ],
        compiler_params=pltpu.CompilerParams(
            dimension_semantics=("arbitrary",)),
        name="qeu_proj",
    )(z2, Wcat, GoT, gb_tile, b_tile, gbo2, lnw2, lnb2)

    CB = 16
    abcd3 = abcd.reshape(4 * C, N, N)
    kt = pl.pallas_call(
        _p2_einsum,
        grid=(C // CB,),
        in_specs=[
            pl.BlockSpec((CB, N, N), lambda o: (o, 0, 0)),
            pl.BlockSpec((CB, N, N), lambda o: (o + C // CB, 0, 0)),
            pl.BlockSpec((CB, N, N), lambda o: (o + 2 * (C // CB), 0, 0)),
            pl.BlockSpec((CB, N, N), lambda o: (o + 3 * (C // CB), 0, 0)),
        ],
        out_specs=pl.BlockSpec((CB, N, N), lambda o: (o, 0, 0)),
        out_shape=jax.ShapeDtypeStruct((C, N, N), jnp.bfloat16),
        compiler_params=pltpu.CompilerParams(
            dimension_semantics=("arbitrary",)),
        name="qeu_einsum",
    )(abcd3, abcd3, abcd3, abcd3)

    Wo2 = (Wo.T * ln_o_w[:, None]).astype(jnp.bfloat16)
    bo2 = (bo + ln_o_b @ Wo.T).reshape(1, D)

    k2 = kt.reshape(C, NN)
    out2 = pl.pallas_call(
        _p3_out,
        grid=(NN // TQ,),
        in_specs=[
            pl.BlockSpec((C, TQ), lambda i: (0, i)),
            pl.BlockSpec((TQ, D), lambda i: (i, 0)),
            pl.BlockSpec((C, D), lambda i: (0, 0)),
            pl.BlockSpec((1, D), lambda i: (0, 0)),
            pl.BlockSpec((1, D), lambda i: (0, 0)),
        ],
        out_specs=pl.BlockSpec((TQ, D), lambda i: (i, 0)),
        out_shape=jax.ShapeDtypeStruct((NN, D), jnp.float32),
        compiler_params=pltpu.CompilerParams(
            dimension_semantics=("arbitrary",)),
        name="qeu_out",
    )(k2, gate, Wo2, bo2, gbo2)

    return out2.reshape(B, N, N, D)
